# Initial kernel scaffold; baseline (speedup 1.0000x reference)
import jax
import jax.numpy as jnp
from jax.experimental import pallas as pl


def kernel(x, edge_index):
    raise NotImplementedError("write your pallas kernel here")


# speedup vs baseline: 9.6783x; 9.6783x over previous
import jax
import jax.numpy as jnp
from jax import lax
from jax.experimental import pallas as pl
from jax.experimental.pallas import tpu as pltpu
from jax.experimental.pallas import tpu_sc as plsc

_N = 10000
_D = 128
_E = 320000
_STEPS = 16
_HH = 64
_NS = 16
_NC = 2
_CHUNK = 128
_EPT = _E // _NS
_NCHUNK = 160
_EPADN = _NCHUNK * _CHUNK - _EPT
_TRASH = 112
_NPAD = _N + _TRASH
_RPT = _N // _NS
_RCH = 125
_NRCH = _RPT // _RCH
_NBUF = 4
_GROUPS = _NCHUNK // _NBUF
_ZPT = _NPAD // _NS
_ZCH = 79
_NZCH = _ZPT // _ZCH


def _body(xs, srcs, dsts, ones_h, zeros_h, yh, wv, cv,
          acc, idx_s, idx_d, bufs, ones, zbuf, gsem, ssem):
    c = lax.axis_index("c")
    s = lax.axis_index("s")
    yh_c = yh.at[c]
    wv_c = wv.at[c]
    cv_c = cv.at[c]
    xs_c = xs.at[c]

    pltpu.sync_copy(srcs.at[s], idx_s)
    pltpu.sync_copy(dsts.at[s], idx_d)
    pltpu.sync_copy(ones_h, ones)
    pltpu.sync_copy(zeros_h, zbuf)

    z0 = s * _ZPT
    for k in range(_NZCH):
        pltpu.sync_copy(zbuf.at[pl.ds(0, _ZCH)],
                        acc.at[pl.ds(z0 + k * _ZCH, _ZCH)])
    plsc.subcore_barrier()

    def deg_body(j, carry):
        pltpu.sync_copy(ones, acc.at[idx_d.at[j]], add=True)
        return carry
    lax.fori_loop(0, _NCHUNK, deg_body, None)
    plsc.subcore_barrier()

    r0 = s * _RPT
    b0 = bufs.at[0]
    b1 = bufs.at[1]
    b2 = bufs.at[2]
    b3 = bufs.at[3]
    for k in range(_NRCH):
        rows = pl.ds(r0 + k * _RCH, _RCH)
        pltpu.sync_copy(acc.at[rows], b0.at[pl.ds(0, _RCH)])
        pltpu.sync_copy(xs_c.at[rows], b1.at[pl.ds(0, _RCH)])
        pltpu.sync_copy(b1.at[pl.ds(0, _RCH)], yh_c.at[rows])

        def init_row(r, carry):
            for jj in range(_HH // 16):
                sl = pl.ds(jj * 16, 16)
                w = 0.5 / (1.0 + b0[r, sl])
                b0[r, sl] = w
                b1[r, sl] = w * b1[r, sl]
            return carry
        lax.fori_loop(0, _RCH, init_row, None)

        pltpu.sync_copy(b0.at[pl.ds(0, _RCH)], wv_c.at[rows])
        pltpu.sync_copy(b1.at[pl.ds(0, _RCH)], cv_c.at[rows])
        pltpu.sync_copy(zbuf.at[pl.ds(0, _RCH)], acc.at[rows])
    plsc.subcore_barrier()

    def step(t, carry):
        def group(g, gcarry):
            gd = []
            for b in range(_NBUF):
                j = g * _NBUF + b
                gd.append(pltpu.async_copy(yh_c.at[idx_s.at[j]],
                                           bufs.at[b], gsem.at[b]))
            sd = []
            for b in range(_NBUF):
                j = g * _NBUF + b
                gd[b].wait()
                sd.append(pltpu.async_copy(bufs.at[b], acc.at[idx_d.at[j]],
                                           ssem.at[b], add=True))
            for b in range(_NBUF):
                sd[b].wait()
            return gcarry
        lax.fori_loop(0, _GROUPS, group, None)
        plsc.subcore_barrier()

        for k in range(_NRCH):
            rows = pl.ds(r0 + k * _RCH, _RCH)
            pltpu.sync_copy(acc.at[rows], b0.at[pl.ds(0, _RCH)])
            pltpu.sync_copy(wv_c.at[rows], b1.at[pl.ds(0, _RCH)])
            pltpu.sync_copy(cv_c.at[rows], b2.at[pl.ds(0, _RCH)])
            pltpu.sync_copy(yh_c.at[rows], b3.at[pl.ds(0, _RCH)])

            def upd_row(r, ucarry):
                for jj in range(_HH // 16):
                    sl = pl.ds(jj * 16, 16)
                    b3[r, sl] = 0.5 * b3[r, sl] + b1[r, sl] * b0[r, sl] \
                        + b2[r, sl]
                return ucarry
            lax.fori_loop(0, _RCH, upd_row, None)

            pltpu.sync_copy(b3.at[pl.ds(0, _RCH)], yh_c.at[rows])
            pltpu.sync_copy(zbuf.at[pl.ds(0, _RCH)], acc.at[rows])
        plsc.subcore_barrier()
        return carry
    lax.fori_loop(0, _STEPS, step, None)


_sc_fn = pl.kernel(
    _body,
    out_type=[
        jax.ShapeDtypeStruct((_NC, _N, _HH), jnp.float32),
        jax.ShapeDtypeStruct((_NC, _N, _HH), jnp.float32),
        jax.ShapeDtypeStruct((_NC, _N, _HH), jnp.float32),
    ],
    mesh=plsc.VectorSubcoreMesh(core_axis_name="c", subcore_axis_name="s"),
    compiler_params=pltpu.CompilerParams(use_tc_tiling_on_sc=False),
    scratch_types=[
        pltpu.VMEM_SHARED((_NPAD, _HH), jnp.float32),
        pltpu.VMEM((_NCHUNK, _CHUNK), jnp.int32),
        pltpu.VMEM((_NCHUNK, _CHUNK), jnp.int32),
        pltpu.VMEM((_NBUF, _CHUNK, _HH), jnp.float32),
        pltpu.VMEM((_CHUNK, _HH), jnp.float32),
        pltpu.VMEM((_CHUNK, _HH), jnp.float32),
        pltpu.SemaphoreType.DMA((_NBUF,)),
        pltpu.SemaphoreType.DMA((_NBUF,)),
    ],
)


def kernel(x, edge_index):
    src = edge_index[0].astype(jnp.int32)
    dst = edge_index[1].astype(jnp.int32)

    src_t = src.reshape(_NS, _EPT)
    dst_t = dst.reshape(_NS, _EPT)
    pad_i = jnp.arange(_EPADN, dtype=jnp.int32)
    pad_src = jnp.broadcast_to((pad_i * 397) % _N, (_NS, _EPADN))
    pad_dst = jnp.broadcast_to(_N + pad_i % _TRASH, (_NS, _EPADN))
    srcs = jnp.concatenate([src_t, pad_src], axis=1)
    srcs = srcs.reshape(_NS, _NCHUNK, _CHUNK)
    dsts = jnp.concatenate([dst_t, pad_dst], axis=1)
    dsts = dsts.reshape(_NS, _NCHUNK, _CHUNK)

    xs = jnp.stack([x[:, :_HH], x[:, _HH:]])
    ones_h = jnp.ones((_CHUNK, _HH), jnp.float32)
    zeros_h = jnp.zeros((_CHUNK, _HH), jnp.float32)

    yh, _wv, _cv = _sc_fn(xs, srcs, dsts, ones_h, zeros_h)
    return jnp.concatenate([yh[0], yh[1]], axis=1)


# speedup vs baseline: 12.8427x; 1.3270x over previous
import jax
import jax.numpy as jnp
from jax import lax
from jax.experimental import pallas as pl
from jax.experimental.pallas import tpu as pltpu
from jax.experimental.pallas import tpu_sc as plsc

_N = 10000
_D = 128
_E = 320000
_STEPS = 16
_HH = 64
_NS = 16
_NC = 2
_CHUNK = 128
_EPT = _E // _NS
_NCHUNK = 160
_EPADN = _NCHUNK * _CHUNK - _EPT
_TRASH = 112
_NPAD = _N + _TRASH
_RPT = _N // _NS
_RCH = 125
_NRCH = _RPT // _RCH
_NBUF = 4
_SETN = 2
_G2 = _NCHUNK // (2 * _SETN)
_ZPT = _NPAD // _NS
_ZCH = 79
_NZCH = _ZPT // _ZCH


def _body(xs, srcs, dsts, ones_h, zeros_h, yh, wv, cv,
          acc, idx_s, idx_d, bufs, gsem, ssem):
    c = lax.axis_index("c")
    s = lax.axis_index("s")
    yh_c = yh.at[c]
    wv_c = wv.at[c]
    cv_c = cv.at[c]
    xs_c = xs.at[c]

    pltpu.sync_copy(srcs.at[s], idx_s)
    pltpu.sync_copy(dsts.at[s], idx_d)
    pltpu.sync_copy(ones_h, bufs.at[0])

    z0 = s * _ZPT
    for k in range(_NZCH):
        pltpu.sync_copy(zeros_h.at[pl.ds(0, _ZCH)],
                        acc.at[pl.ds(z0 + k * _ZCH, _ZCH)])
    plsc.subcore_barrier()

    def deg_body(j, carry):
        pltpu.sync_copy(bufs.at[0], acc.at[idx_d.at[j]], add=True)
        return carry
    lax.fori_loop(0, _NCHUNK, deg_body, None)
    plsc.subcore_barrier()

    r0 = s * _RPT
    b0 = bufs.at[0]
    b1 = bufs.at[1]
    b2 = bufs.at[2]
    b3 = bufs.at[3]
    for k in range(_NRCH):
        rows = pl.ds(r0 + k * _RCH, _RCH)
        pltpu.sync_copy(acc.at[rows], b0.at[pl.ds(0, _RCH)])
        pltpu.sync_copy(xs_c.at[rows], b1.at[pl.ds(0, _RCH)])
        pltpu.sync_copy(b1.at[pl.ds(0, _RCH)], yh_c.at[rows])

        def init_row(r, carry):
            for jj in range(_HH // 16):
                sl = pl.ds(jj * 16, 16)
                w = 0.5 / (1.0 + b0[r, sl])
                b0[r, sl] = w
                b1[r, sl] = w * b1[r, sl]
            return carry
        lax.fori_loop(0, _RCH, init_row, None)

        pltpu.sync_copy(b0.at[pl.ds(0, _RCH)], wv_c.at[rows])
        pltpu.sync_copy(b1.at[pl.ds(0, _RCH)], cv_c.at[rows])
        pltpu.sync_copy(zeros_h.at[pl.ds(0, _RCH)], acc.at[rows])
    plsc.subcore_barrier()

    def step(t, carry):
        for b in range(_SETN):
            pltpu.async_copy(yh_c.at[idx_s.at[b]], bufs.at[b], gsem.at[b])

        def group2(gg, gcarry):
            g1 = []
            for b in range(_SETN):
                j = (2 * gg + 1) * _SETN + b
                g1.append(pltpu.async_copy(yh_c.at[idx_s.at[j]],
                                           bufs.at[_SETN + b],
                                           gsem.at[_SETN + b]))
            s0 = []
            for b in range(_SETN):
                j = (2 * gg) * _SETN + b
                pltpu.make_async_copy(yh_c.at[pl.ds(0, _CHUNK)],
                                      bufs.at[b], gsem.at[b]).wait()
                s0.append(pltpu.async_copy(bufs.at[b], acc.at[idx_d.at[j]],
                                           ssem.at[b], add=True))
            for d in s0:
                d.wait()

            @pl.when(gg < _G2 - 1)
            def _refill():
                for b in range(_SETN):
                    j = (2 * gg + 2) * _SETN + b
                    pltpu.async_copy(yh_c.at[idx_s.at[j]],
                                     bufs.at[b], gsem.at[b])

            s1 = []
            for b in range(_SETN):
                j = (2 * gg + 1) * _SETN + b
                g1[b].wait()
                s1.append(pltpu.async_copy(bufs.at[_SETN + b],
                                           acc.at[idx_d.at[j]],
                                           ssem.at[_SETN + b], add=True))
            for d in s1:
                d.wait()
            return gcarry
        lax.fori_loop(0, _G2, group2, None)
        plsc.subcore_barrier()

        for k in range(_NRCH):
            rows = pl.ds(r0 + k * _RCH, _RCH)
            da = pltpu.async_copy(acc.at[rows], b0.at[pl.ds(0, _RCH)],
                                  gsem.at[0])
            dw = pltpu.async_copy(wv_c.at[rows], b1.at[pl.ds(0, _RCH)],
                                  gsem.at[1])
            dc = pltpu.async_copy(cv_c.at[rows], b2.at[pl.ds(0, _RCH)],
                                  gsem.at[2])
            dy = pltpu.async_copy(yh_c.at[rows], b3.at[pl.ds(0, _RCH)],
                                  gsem.at[3])
            da.wait()
            dw.wait()
            dc.wait()
            dy.wait()

            def upd_row(r, ucarry):
                for jj in range(_HH // 16):
                    sl = pl.ds(jj * 16, 16)
                    b3[r, sl] = 0.5 * b3[r, sl] + b1[r, sl] * b0[r, sl] \
                        + b2[r, sl]
                return ucarry
            lax.fori_loop(0, _RCH, upd_row, None)

            pltpu.sync_copy(b3.at[pl.ds(0, _RCH)], yh_c.at[rows])
            pltpu.sync_copy(zeros_h.at[pl.ds(0, _RCH)], acc.at[rows])
        plsc.subcore_barrier()
        return carry
    lax.fori_loop(0, _STEPS, step, None)


_sc_fn = pl.kernel(
    _body,
    out_type=[
        jax.ShapeDtypeStruct((_NC, _N, _HH), jnp.float32),
        jax.ShapeDtypeStruct((_NC, _N, _HH), jnp.float32),
        jax.ShapeDtypeStruct((_NC, _N, _HH), jnp.float32),
    ],
    mesh=plsc.VectorSubcoreMesh(core_axis_name="c", subcore_axis_name="s"),
    compiler_params=pltpu.CompilerParams(use_tc_tiling_on_sc=False),
    scratch_types=[
        pltpu.VMEM_SHARED((_NPAD, _HH), jnp.float32),
        pltpu.VMEM((_NCHUNK, _CHUNK), jnp.int32),
        pltpu.VMEM((_NCHUNK, _CHUNK), jnp.int32),
        pltpu.VMEM((_NBUF, _CHUNK, _HH), jnp.float32),
        pltpu.SemaphoreType.DMA((_NBUF,)),
        pltpu.SemaphoreType.DMA((_NBUF,)),
    ],
)


def kernel(x, edge_index):
    src = edge_index[0].astype(jnp.int32)
    dst = edge_index[1].astype(jnp.int32)

    src_t = src.reshape(_NS, _EPT)
    dst_t = dst.reshape(_NS, _EPT)
    pad_i = jnp.arange(_EPADN, dtype=jnp.int32)
    pad_src = jnp.broadcast_to((pad_i * 397) % _N, (_NS, _EPADN))
    pad_dst = jnp.broadcast_to(_N + pad_i % _TRASH, (_NS, _EPADN))
    srcs = jnp.concatenate([src_t, pad_src], axis=1)
    srcs = srcs.reshape(_NS, _NCHUNK, _CHUNK)
    dsts = jnp.concatenate([dst_t, pad_dst], axis=1)
    dsts = dsts.reshape(_NS, _NCHUNK, _CHUNK)

    xs = jnp.stack([x[:, :_HH], x[:, _HH:]])
    ones_h = jnp.ones((_CHUNK, _HH), jnp.float32)
    zeros_h = jnp.zeros((_CHUNK, _HH), jnp.float32)

    yh, _wv, _cv = _sc_fn(xs, srcs, dsts, ones_h, zeros_h)
    return jnp.concatenate([yh[0], yh[1]], axis=1)


# speedup vs baseline: 12.9819x; 1.0108x over previous
import jax
import jax.numpy as jnp
from jax import lax
from jax.experimental import pallas as pl
from jax.experimental.pallas import tpu as pltpu
from jax.experimental.pallas import tpu_sc as plsc

_N = 10000
_D = 128
_E = 320000
_STEPS = 16
_HH = 64
_NS = 16
_NC = 2
_SCH = 128
_GCH = 256
_EPT = _E // _NS
_NSCH = 160
_NGCH = 80
_EPADN = _NSCH * _SCH - _EPT
_TRASH = 112
_NPAD = _N + _TRASH
_RPT = _N // _NS
_RCH = 125
_NRCH = _RPT // _RCH
_G2 = _NGCH // 2
_ZPT = _NPAD // _NS
_ZCH = 79
_NZCH = _ZPT // _ZCH
_O = (0, 128, 256, 384)


def _body(xs, srcs, dsts, ones_h, zeros_h, yh, wv, cv,
          acc, idx_s, idx_d, bufs, gsem, ssem):
    c = lax.axis_index("c")
    s = lax.axis_index("s")
    yh_c = yh.at[c]
    wv_c = wv.at[c]
    cv_c = cv.at[c]
    xs_c = xs.at[c]

    pltpu.sync_copy(srcs.at[s], idx_s)
    pltpu.sync_copy(dsts.at[s], idx_d)
    pltpu.sync_copy(ones_h, bufs.at[pl.ds(0, _SCH)])

    z0 = s * _ZPT
    for k in range(_NZCH):
        pltpu.sync_copy(zeros_h.at[pl.ds(0, _ZCH)],
                        acc.at[pl.ds(z0 + k * _ZCH, _ZCH)])
    plsc.subcore_barrier()

    def deg_body(j, carry):
        pltpu.sync_copy(bufs.at[pl.ds(0, _SCH)], acc.at[idx_d.at[j]],
                        add=True)
        return carry
    lax.fori_loop(0, _NSCH, deg_body, None)
    plsc.subcore_barrier()

    r0 = s * _RPT
    for k in range(_NRCH):
        rows = pl.ds(r0 + k * _RCH, _RCH)
        pltpu.sync_copy(acc.at[rows], bufs.at[pl.ds(_O[0], _RCH)])
        pltpu.sync_copy(xs_c.at[rows], bufs.at[pl.ds(_O[1], _RCH)])
        pltpu.sync_copy(bufs.at[pl.ds(_O[1], _RCH)], yh_c.at[rows])

        def init_row(r, carry):
            for jj in range(_HH // 16):
                sl = pl.ds(jj * 16, 16)
                w = 0.5 / (1.0 + bufs[_O[0] + r, sl])
                bufs[_O[0] + r, sl] = w
                bufs[_O[1] + r, sl] = w * bufs[_O[1] + r, sl]
            return carry
        lax.fori_loop(0, _RCH, init_row, None)

        pltpu.sync_copy(bufs.at[pl.ds(_O[0], _RCH)], wv_c.at[rows])
        pltpu.sync_copy(bufs.at[pl.ds(_O[1], _RCH)], cv_c.at[rows])
        pltpu.sync_copy(zeros_h.at[pl.ds(0, _RCH)], acc.at[rows])
    plsc.subcore_barrier()

    def step(t, carry):
        pltpu.async_copy(yh_c.at[idx_s.at[0]],
                         bufs.at[pl.ds(0, _GCH)], gsem.at[0])

        def group2(gg, gcarry):
            g1 = pltpu.async_copy(yh_c.at[idx_s.at[2 * gg + 1]],
                                  bufs.at[pl.ds(_GCH, _GCH)], gsem.at[1])
            pltpu.make_async_copy(yh_c.at[pl.ds(0, _GCH)],
                                  bufs.at[pl.ds(0, _GCH)], gsem.at[0]).wait()
            s0 = []
            for q in range(2):
                j = 4 * gg + q
                s0.append(pltpu.async_copy(
                    bufs.at[pl.ds(q * _SCH, _SCH)],
                    acc.at[idx_d.at[j]], ssem.at[q], add=True))
            for d in s0:
                d.wait()

            @pl.when(gg < _G2 - 1)
            def _refill():
                pltpu.async_copy(yh_c.at[idx_s.at[2 * gg + 2]],
                                 bufs.at[pl.ds(0, _GCH)], gsem.at[0])

            g1.wait()
            s1 = []
            for q in range(2):
                j = 4 * gg + 2 + q
                s1.append(pltpu.async_copy(
                    bufs.at[pl.ds(_GCH + q * _SCH, _SCH)],
                    acc.at[idx_d.at[j]], ssem.at[2 + q], add=True))
            for d in s1:
                d.wait()
            return gcarry
        lax.fori_loop(0, _G2, group2, None)
        plsc.subcore_barrier()

        for k in range(_NRCH):
            rows = pl.ds(r0 + k * _RCH, _RCH)
            da = pltpu.async_copy(acc.at[rows],
                                  bufs.at[pl.ds(_O[0], _RCH)], gsem.at[0])
            dw = pltpu.async_copy(wv_c.at[rows],
                                  bufs.at[pl.ds(_O[1], _RCH)], gsem.at[1])
            dc = pltpu.async_copy(cv_c.at[rows],
                                  bufs.at[pl.ds(_O[2], _RCH)], ssem.at[0])
            dy = pltpu.async_copy(yh_c.at[rows],
                                  bufs.at[pl.ds(_O[3], _RCH)], ssem.at[1])
            da.wait()
            dw.wait()
            dc.wait()
            dy.wait()

            def upd_row(r, ucarry):
                for jj in range(_HH // 16):
                    sl = pl.ds(jj * 16, 16)
                    bufs[_O[3] + r, sl] = (
                        0.5 * bufs[_O[3] + r, sl]
                        + bufs[_O[1] + r, sl] * bufs[_O[0] + r, sl]
                        + bufs[_O[2] + r, sl])
                return ucarry
            lax.fori_loop(0, _RCH, upd_row, None)

            pltpu.sync_copy(bufs.at[pl.ds(_O[3], _RCH)], yh_c.at[rows])
            pltpu.sync_copy(zeros_h.at[pl.ds(0, _RCH)], acc.at[rows])
        plsc.subcore_barrier()
        return carry
    lax.fori_loop(0, _STEPS, step, None)


_sc_fn = pl.kernel(
    _body,
    out_type=[
        jax.ShapeDtypeStruct((_NC, _N, _HH), jnp.float32),
        jax.ShapeDtypeStruct((_NC, _N, _HH), jnp.float32),
        jax.ShapeDtypeStruct((_NC, _N, _HH), jnp.float32),
    ],
    mesh=plsc.VectorSubcoreMesh(core_axis_name="c", subcore_axis_name="s"),
    compiler_params=pltpu.CompilerParams(use_tc_tiling_on_sc=False),
    scratch_types=[
        pltpu.VMEM_SHARED((_NPAD, _HH), jnp.float32),
        pltpu.VMEM((_NGCH, _GCH), jnp.int32),
        pltpu.VMEM((_NSCH, _SCH), jnp.int32),
        pltpu.VMEM((2 * _GCH, _HH), jnp.float32),
        pltpu.SemaphoreType.DMA((2,)),
        pltpu.SemaphoreType.DMA((4,)),
    ],
)


def kernel(x, edge_index):
    src = edge_index[0].astype(jnp.int32)
    dst = edge_index[1].astype(jnp.int32)

    src_t = src.reshape(_NS, _EPT)
    dst_t = dst.reshape(_NS, _EPT)
    pad_i = jnp.arange(_EPADN, dtype=jnp.int32)
    pad_src = jnp.broadcast_to((pad_i * 397) % _N, (_NS, _EPADN))
    pad_dst = jnp.broadcast_to(_N + pad_i % _TRASH, (_NS, _EPADN))
    srcs = jnp.concatenate([src_t, pad_src], axis=1)
    srcs = srcs.reshape(_NS, _NGCH, _GCH)
    dsts = jnp.concatenate([dst_t, pad_dst], axis=1)
    dsts = dsts.reshape(_NS, _NSCH, _SCH)

    xs = jnp.stack([x[:, :_HH], x[:, _HH:]])
    ones_h = jnp.ones((_SCH, _HH), jnp.float32)
    zeros_h = jnp.zeros((_SCH, _HH), jnp.float32)

    yh, _wv, _cv = _sc_fn(xs, srcs, dsts, ones_h, zeros_h)
    return jnp.concatenate([yh[0], yh[1]], axis=1)


# speedup vs baseline: 14.0875x; 1.0852x over previous
import jax
import jax.numpy as jnp
from jax import lax
from jax.experimental import pallas as pl
from jax.experimental.pallas import tpu as pltpu
from jax.experimental.pallas import tpu_sc as plsc

_N = 10000
_D = 128
_E = 320000
_STEPS = 16
_HH = 64
_NS = 16
_NC = 2
_SCH = 128
_GCH = 256
_EPT = _E // _NS
_NSCH = 160
_NGCH = 80
_EPADN = _NSCH * _SCH - _EPT
_TRASH = 112
_NPAD = _N + _TRASH
_RPT = _N // _NS
_RCH = 125
_NRCH = _RPT // _RCH
_G2 = _NGCH // 2
_ZPT = _NPAD // _NS
_ZCH = 79
_NZCH = _ZPT // _ZCH
_O = (0, 128, 256, 384)


def _body(xs, srcs, dsts, ones_h, zeros_h, yh, wv,
          acc, idx_s, idx_d, bufs, gsem, ssem):
    c = lax.axis_index("c")
    s = lax.axis_index("s")
    yh_c = yh.at[c]
    wv_c = wv.at[c]
    xs_c = xs.at[c]

    pltpu.sync_copy(srcs.at[s], idx_s)
    pltpu.sync_copy(dsts.at[s], idx_d)
    pltpu.sync_copy(ones_h, bufs.at[pl.ds(0, _SCH)])

    z0 = s * _ZPT
    for k in range(_NZCH):
        pltpu.sync_copy(zeros_h.at[pl.ds(0, _ZCH)],
                        acc.at[pl.ds(z0 + k * _ZCH, _ZCH)])
    plsc.subcore_barrier()

    def deg_body(i, carry):
        sd = []
        for q in range(4):
            sd.append(pltpu.async_copy(bufs.at[pl.ds(0, _SCH)],
                                       acc.at[idx_d.at[4 * i + q]],
                                       ssem.at[q], add=True))
        for d in sd:
            d.wait()
        return carry
    lax.fori_loop(0, _NSCH // 4, deg_body, None)
    plsc.subcore_barrier()

    r0 = s * _RPT
    for k in range(_NRCH):
        rows = pl.ds(r0 + k * _RCH, _RCH)
        pltpu.sync_copy(acc.at[rows], bufs.at[pl.ds(_O[0], _RCH)])

        def init_row(r, carry):
            for jj in range(_HH // 16):
                sl = pl.ds(jj * 16, 16)
                bufs[_O[0] + r, sl] = 0.5 / (1.0 + bufs[_O[0] + r, sl])
            return carry
        lax.fori_loop(0, _RCH, init_row, None)

        pltpu.sync_copy(bufs.at[pl.ds(_O[0], _RCH)], wv_c.at[rows])
        pltpu.sync_copy(xs_c.at[rows], bufs.at[pl.ds(_O[1], _RCH)])
        pltpu.sync_copy(bufs.at[pl.ds(_O[1], _RCH)], yh_c.at[rows])
        pltpu.sync_copy(bufs.at[pl.ds(_O[1], _RCH)], acc.at[rows])
    plsc.subcore_barrier()

    def step(t, carry):
        pltpu.async_copy(yh_c.at[idx_s.at[0]],
                         bufs.at[pl.ds(0, _GCH)], gsem.at[0])

        def group2(gg, gcarry):
            g1 = pltpu.async_copy(yh_c.at[idx_s.at[2 * gg + 1]],
                                  bufs.at[pl.ds(_GCH, _GCH)], gsem.at[1])
            pltpu.make_async_copy(yh_c.at[pl.ds(0, _GCH)],
                                  bufs.at[pl.ds(0, _GCH)], gsem.at[0]).wait()
            s0 = []
            for q in range(2):
                j = 4 * gg + q
                s0.append(pltpu.async_copy(
                    bufs.at[pl.ds(q * _SCH, _SCH)],
                    acc.at[idx_d.at[j]], ssem.at[q], add=True))
            for d in s0:
                d.wait()

            @pl.when(gg < _G2 - 1)
            def _refill():
                pltpu.async_copy(yh_c.at[idx_s.at[2 * gg + 2]],
                                 bufs.at[pl.ds(0, _GCH)], gsem.at[0])

            g1.wait()
            s1 = []
            for q in range(2):
                j = 4 * gg + 2 + q
                s1.append(pltpu.async_copy(
                    bufs.at[pl.ds(_GCH + q * _SCH, _SCH)],
                    acc.at[idx_d.at[j]], ssem.at[2 + q], add=True))
            for d in s1:
                d.wait()
            return gcarry
        lax.fori_loop(0, _G2, group2, None)
        plsc.subcore_barrier()

        for k in range(_NRCH):
            rows = pl.ds(r0 + k * _RCH, _RCH)
            if k > 0:
                pltpu.make_async_copy(bufs.at[pl.ds(_O[3], _RCH)],
                                      yh_c.at[rows], ssem.at[2]).wait()
            da = pltpu.async_copy(acc.at[rows],
                                  bufs.at[pl.ds(_O[0], _RCH)], gsem.at[0])
            dw = pltpu.async_copy(wv_c.at[rows],
                                  bufs.at[pl.ds(_O[1], _RCH)], gsem.at[1])
            dy = pltpu.async_copy(yh_c.at[rows],
                                  bufs.at[pl.ds(_O[3], _RCH)], ssem.at[0])
            da.wait()
            if k > 0:
                pltpu.make_async_copy(xs_c.at[rows], acc.at[rows],
                                      ssem.at[3]).wait()
            pltpu.async_copy(xs_c.at[rows], acc.at[rows], ssem.at[3])
            dw.wait()
            dy.wait()

            def upd_row(r, ucarry):
                for jj in range(_HH // 16):
                    sl = pl.ds(jj * 16, 16)
                    bufs[_O[3] + r, sl] = (
                        0.5 * bufs[_O[3] + r, sl]
                        + bufs[_O[1] + r, sl] * bufs[_O[0] + r, sl])
                return ucarry
            lax.fori_loop(0, _RCH, upd_row, None)

            pltpu.async_copy(bufs.at[pl.ds(_O[3], _RCH)], yh_c.at[rows],
                             ssem.at[2])
        pltpu.make_async_copy(bufs.at[pl.ds(_O[3], _RCH)],
                              yh_c.at[pl.ds(r0, _RCH)], ssem.at[2]).wait()
        pltpu.make_async_copy(xs_c.at[pl.ds(r0, _RCH)],
                              acc.at[pl.ds(r0, _RCH)], ssem.at[3]).wait()
        plsc.subcore_barrier()
        return carry
    lax.fori_loop(0, _STEPS, step, None)


_sc_fn = pl.kernel(
    _body,
    out_type=[
        jax.ShapeDtypeStruct((_NC, _N, _HH), jnp.float32),
        jax.ShapeDtypeStruct((_NC, _N, _HH), jnp.float32),
    ],
    mesh=plsc.VectorSubcoreMesh(core_axis_name="c", subcore_axis_name="s"),
    compiler_params=pltpu.CompilerParams(use_tc_tiling_on_sc=False),
    scratch_types=[
        pltpu.VMEM_SHARED((_NPAD, _HH), jnp.float32),
        pltpu.VMEM((_NGCH, _GCH), jnp.int32),
        pltpu.VMEM((_NSCH, _SCH), jnp.int32),
        pltpu.VMEM((2 * _GCH, _HH), jnp.float32),
        pltpu.SemaphoreType.DMA((2,)),
        pltpu.SemaphoreType.DMA((4,)),
    ],
)


def kernel(x, edge_index):
    src = edge_index[0].astype(jnp.int32)
    dst = edge_index[1].astype(jnp.int32)

    src_t = src.reshape(_NS, _EPT)
    dst_t = dst.reshape(_NS, _EPT)
    pad_i = jnp.arange(_EPADN, dtype=jnp.int32)
    pad_src = jnp.broadcast_to((pad_i * 397) % _N, (_NS, _EPADN))
    pad_dst = jnp.broadcast_to(_N + pad_i % _TRASH, (_NS, _EPADN))
    srcs = jnp.concatenate([src_t, pad_src], axis=1)
    srcs = srcs.reshape(_NS, _NGCH, _GCH)
    dsts = jnp.concatenate([dst_t, pad_dst], axis=1)
    dsts = dsts.reshape(_NS, _NSCH, _SCH)

    xs = jnp.stack([x[:, :_HH], x[:, _HH:]])
    ones_h = jnp.ones((_SCH, _HH), jnp.float32)
    zeros_h = jnp.zeros((_SCH, _HH), jnp.float32)

    yh, _wv = _sc_fn(xs, srcs, dsts, ones_h, zeros_h)
    return jnp.concatenate([yh[0], yh[1]], axis=1)
